# baseline (speedup 1.0000x reference)
import functools

import jax
import jax.numpy as jnp
from jax import lax
from jax.experimental import pallas as pl
from jax.experimental.pallas import tpu as pltpu
from jax.experimental.pallas import tpu_sc as plsc

_NC = 2
_NS = 16
_CH = 128
_ZR = 625


def _sc_fill_scatter(idx3d, upd3d, M):
    NC_, N, D = upd3d.shape
    per_w = N // _NS
    n_ch = per_w // _CH
    half = M // _NC
    rows_w = half // _NS
    n_z = rows_w // _ZR

    mesh = plsc.VectorSubcoreMesh(
        core_axis_name="c", subcore_axis_name="s",
        num_cores=_NC, num_subcores=_NS,
    )

    @functools.partial(
        pl.kernel,
        out_type=jax.ShapeDtypeStruct((M, D), jnp.float32),
        mesh=mesh,
        compiler_params=pltpu.CompilerParams(use_tc_tiling_on_sc=False),
        scratch_types=[
            pltpu.VMEM((_ZR, D), jnp.float32),
            pltpu.VMEM((n_ch, _CH), jnp.int32),
            pltpu.VMEM((per_w, D), jnp.float32),
            pltpu.SemaphoreType.DMA,
            pltpu.SemaphoreType.DMA,
        ],
    )
    def fill_scatter(idx_hbm, upd_hbm, out_hbm, zbuf, idxv, updv, zsem, lsem):
        c = lax.axis_index("c")
        s = lax.axis_index("s")

        @pl.loop(0, _ZR)
        def _(r):
            for t in range(D // 16):
                zbuf[r, pl.ds(t * 16, 16)] = jnp.zeros((16,), jnp.float32)

        base = c * half + s * rows_w
        copies = [
            pltpu.async_copy(
                zbuf, out_hbm.at[pl.ds(base + k * _ZR, _ZR), :], zsem
            )
            for k in range(n_z)
        ]

        cp_i = pltpu.async_copy(idx_hbm.at[c, pl.ds(s * n_ch, n_ch)], idxv, lsem)
        cp_u = pltpu.async_copy(upd_hbm.at[c, pl.ds(s * per_w, per_w)], updv, lsem)
        cp_i.wait()
        cp_u.wait()
        for cp in copies:
            cp.wait()

        plsc.subcore_barrier()

        for j in range(n_ch):
            pltpu.sync_copy(
                updv.at[pl.ds(j * _CH, _CH)], out_hbm.at[idxv.at[j]]
            )

    return fill_scatter(idx3d, upd3d)


_BR = 1024


def _tr_body(x_ref, o_ref):
    D = o_ref.shape[0]
    x2 = x_ref[...].reshape(_BR // 2, 2 * D)
    y = x2.T
    st = jnp.stack([y[:D, :], y[D:, :]], axis=2)
    o_ref[...] = st.reshape(D, _BR)


def _tc_transpose(l1d, M, D):
    return pl.pallas_call(
        _tr_body,
        grid=((M + _BR - 1) // _BR,),
        in_specs=[pl.BlockSpec((_BR * D,), lambda i: (i,))],
        out_specs=pl.BlockSpec((D, _BR), lambda i: (0, i)),
        out_shape=jax.ShapeDtypeStruct((D, M), jnp.float32),
    )(l1d)


def kernel(target_g, original_level_h, original_level_idx):
    M = target_g.shape[0]
    N, D = original_level_h.shape
    idx = original_level_idx.astype(jnp.int32)

    pos = jnp.arange(N, dtype=jnp.int32)
    tick = jnp.zeros((M,), jnp.int32).at[idx].max(pos + 1)
    winner = tick[idx] - 1
    upd = original_level_h[winner]

    half = M // _NC
    idx_c, upd_c = [], []
    for c in range(_NC):
        in_half = (idx >= c * half) & (idx < (c + 1) * half)
        first = jnp.argmax(in_half)
        has = in_half[first]
        pad_idx = jnp.where(has, idx[first], c * half)
        pad_val = jnp.where(has, upd[first], jnp.zeros((D,), upd.dtype))
        idx_c.append(jnp.where(in_half, idx, pad_idx))
        upd_c.append(jnp.where(in_half[:, None], upd, pad_val[None, :]))
    idx3d = jnp.stack(idx_c).reshape(_NC, N // _CH, _CH)
    upd3d = jnp.stack(upd_c)

    lin = _sc_fill_scatter(idx3d, upd3d, M)
    out_t = _tc_transpose(lin.reshape(M * D), M, D)
    return out_t.T


# speedup vs baseline: 36.2084x; 36.2084x over previous
import functools

import jax
import jax.numpy as jnp
from jax import lax
from jax.experimental import pallas as pl
from jax.experimental.pallas import tpu as pltpu
from jax.experimental.pallas import tpu_sc as plsc

_NC = 2
_NS = 16
_NW = _NC * _NS
_CH = 128
_BR = 8192


def _sc_scatter(idx3d, upd2, M):
    N = idx3d.shape[0] * idx3d.shape[1]
    W = upd2.shape[1]
    per_w = N // _NW
    n_ch = per_w // _CH

    mesh = plsc.VectorSubcoreMesh(
        core_axis_name="c", subcore_axis_name="s",
        num_cores=_NC, num_subcores=_NS,
    )

    @functools.partial(
        pl.kernel,
        out_type=jax.ShapeDtypeStruct((M, W), jnp.float32),
        mesh=mesh,
        scratch_types=[
            pltpu.VMEM((n_ch, _CH), jnp.int32),
            pltpu.VMEM((per_w, W), jnp.float32),
            pltpu.SemaphoreType.DMA,
        ],
    )
    def scatter_kernel(idx_hbm, upd_hbm, out_hbm, idxv, updv, sem):
        wid = lax.axis_index("s") * _NC + lax.axis_index("c")
        cp_i = pltpu.async_copy(idx_hbm.at[pl.ds(wid * n_ch, n_ch)], idxv, sem)
        cp_u = pltpu.async_copy(upd_hbm.at[pl.ds(wid * per_w, per_w)], updv, sem)
        cp_i.wait()
        cp_u.wait()
        for j in range(n_ch):
            pltpu.sync_copy(
                updv.at[pl.ds(j * _CH, _CH)], out_hbm.at[idxv.at[j]]
            )

    return scatter_kernel(idx3d, upd2)


def _tr_body(x_ref, t_ref, o_ref):
    y = x_ref[...][:, :64].T
    mask = (t_ref[...] > 0)[None, :]
    o_ref[...] = jnp.where(mask, y, 0.0)


def _tc_finalize(l5, tick, M, D):
    grid = (M + _BR - 1) // _BR
    return pl.pallas_call(
        _tr_body,
        grid=(grid,),
        in_specs=[
            pl.BlockSpec((_BR, 128), lambda i: (i, 0)),
            pl.BlockSpec((_BR,), lambda i: (i,)),
        ],
        out_specs=pl.BlockSpec((D, _BR), lambda i: (0, i)),
        out_shape=jax.ShapeDtypeStruct((D, M), jnp.float32),
    )(l5, tick)


def kernel(target_g, original_level_h, original_level_idx):
    M = target_g.shape[0]
    N, D = original_level_h.shape
    idx = original_level_idx.astype(jnp.int32)

    pos = jnp.arange(N, dtype=jnp.int32)
    tick = jnp.zeros((M,), jnp.int32).at[idx].max(pos + 1)
    winner = tick[idx] - 1
    upd = original_level_h[winner]

    upd2 = jnp.concatenate([upd, upd], axis=1)
    idx3d = idx.reshape(N // _CH, _CH)

    l5 = _sc_scatter(idx3d, upd2, M)
    out_t = _tc_finalize(l5, tick, M, D)
    return out_t.T


# speedup vs baseline: 37.3307x; 1.0310x over previous
import functools

import jax
import jax.numpy as jnp
from jax import lax
from jax.experimental import pallas as pl
from jax.experimental.pallas import tpu as pltpu
from jax.experimental.pallas import tpu_sc as plsc

_NC = 2
_NS = 16
_NW = _NC * _NS
_CH = 128
_BR = 16384


def _sc_scatter(idx3d, upd2, M):
    N, W = upd2.shape
    per_w = N // _NW
    n_ch = per_w // _CH

    mesh = plsc.VectorSubcoreMesh(
        core_axis_name="c", subcore_axis_name="s",
        num_cores=_NC, num_subcores=_NS,
    )

    @functools.partial(
        pl.kernel,
        out_type=jax.ShapeDtypeStruct((M, W), jnp.float32),
        mesh=mesh,
        scratch_types=[
            pltpu.VMEM((n_ch, _CH), jnp.int32),
            pltpu.VMEM((per_w, W), jnp.float32),
            pltpu.SemaphoreType.DMA,
        ],
    )
    def scatter_kernel(idx_hbm, upd_hbm, out_hbm, idxv, updv, sem):
        wid = lax.axis_index("s") * _NC + lax.axis_index("c")
        cp_i = pltpu.async_copy(idx_hbm.at[pl.ds(wid * n_ch, n_ch)], idxv, sem)
        cp_u = pltpu.async_copy(upd_hbm.at[pl.ds(wid * per_w, per_w)], updv, sem)
        cp_i.wait()
        cp_u.wait()
        for j in range(n_ch):
            pltpu.sync_copy(
                updv.at[pl.ds(j * _CH, _CH)], out_hbm.at[idxv.at[j]]
            )

    return scatter_kernel(idx3d, upd2)


def _tr_body(x_ref, t_ref, o_ref):
    y = x_ref[...][:, :64].T
    mask = (t_ref[...] > 0)[None, :]
    o_ref[...] = jnp.where(mask, y, 0.0)


def _tc_finalize(l5, tick, M, D):
    grid = (M + _BR - 1) // _BR
    return pl.pallas_call(
        _tr_body,
        grid=(grid,),
        in_specs=[
            pl.BlockSpec((_BR, 128), lambda i: (i, 0)),
            pl.BlockSpec((_BR,), lambda i: (i,)),
        ],
        out_specs=pl.BlockSpec((D, _BR), lambda i: (0, i)),
        out_shape=jax.ShapeDtypeStruct((D, M), jnp.float32),
    )(l5, tick)


def kernel(target_g, original_level_h, original_level_idx):
    M = target_g.shape[0]
    N, D = original_level_h.shape
    idx = original_level_idx.astype(jnp.int32)

    pos = jnp.arange(N, dtype=jnp.int32)
    tick = jnp.zeros((M,), jnp.int32).at[idx].max(pos + 1)
    winner = tick[idx] - 1
    upd = original_level_h[winner]

    upd2 = jnp.concatenate([upd, upd], axis=1)
    idx3d = idx.reshape(N // _CH, _CH)

    l5 = _sc_scatter(idx3d, upd2, M)
    out_t = _tc_finalize(l5, tick, M, D)
    return out_t.T
